# Initial kernel scaffold; baseline (speedup 1.0000x reference)
import jax
import jax.numpy as jnp
from jax.experimental import pallas as pl


def kernel(edge_index, adj_values, dEmbeds, gEmbeds, dHyper, gHyper, keepRate):
    raise NotImplementedError("write your pallas kernel here")


# speedup vs baseline: 4.7748x; 4.7748x over previous
import functools

import jax
import jax.numpy as jnp
from jax import lax
from jax.experimental import pallas as pl
from jax.experimental.pallas import tpu as pltpu
from jax.experimental.pallas import tpu_sc as plsc

DRUG = 2000
GENE = 8000
N = DRUG + GENE
LATDIM = 128
E = 320000

NC = 2
NS = 16
NW = NC * NS
EPW = E // NW
CH = 80
NCHUNK = EPW // CH
STRIPE = 640
SBLK = 80


def _sc_spmm_body(col_hbm, row_hbm, val_hbm, prev_hbm, out_hbm,
                  idx_all, ridx_all, val_c, rows, acc, sem):
    cid = lax.axis_index("c")
    sid = lax.axis_index("s")
    wid = cid * NS + sid

    def _zero_row(i, _):
        for s8 in range(8):
            rows[i, pl.ds(s8 * 16, 16)] = jnp.zeros((16,), jnp.float32)
        return 0
    lax.fori_loop(0, SBLK, _zero_row, 0)
    for j in range(STRIPE // SBLK):
        @pl.when(jnp.logical_or(sid < NS - 1, j < 5))
        def _():
            pltpu.sync_copy(rows, acc.at[pl.ds(sid * STRIPE + j * SBLK,
                                               SBLK)])
    plsc.subcore_barrier()

    pltpu.sync_copy(col_hbm.at[wid], idx_all)
    pltpu.sync_copy(row_hbm.at[wid], ridx_all)

    def _chunk(ch, _):
        pltpu.sync_copy(val_hbm.at[pl.ds(wid * EPW + ch * CH, CH)], val_c)
        pltpu.async_copy(prev_hbm.at[idx_all.at[ch]], rows, sem).wait()

        def _scale(g, _):
            for u in range(4):
                e = g * 4 + u
                vv = plsc.load_gather(
                    val_c, [jnp.full((16,), e, jnp.int32)])
                for s8 in range(8):
                    rows[e, pl.ds(s8 * 16, 16)] = (
                        rows[e, pl.ds(s8 * 16, 16)] * vv)
            return 0
        lax.fori_loop(0, CH // 4, _scale, 0)

        pltpu.sync_copy(rows, acc.at[ridx_all.at[ch]], add=True)
        return 0
    lax.fori_loop(0, NCHUNK, _chunk, 0)

    plsc.subcore_barrier()
    for j in range(STRIPE // SBLK):
        @pl.when(jnp.logical_or(sid < NS - 1, j < 5))
        def _():
            r0 = sid * STRIPE + j * SBLK
            pltpu.sync_copy(acc.at[pl.ds(r0, SBLK)],
                            out_hbm.at[cid, pl.ds(r0, SBLK)])


_sc_spmm = functools.partial(
    pl.kernel,
    out_type=jax.ShapeDtypeStruct((NC, N, LATDIM), jnp.float32),
    mesh=plsc.VectorSubcoreMesh(core_axis_name="c", subcore_axis_name="s"),
    compiler_params=pltpu.CompilerParams(needs_layout_passes=False),
    scratch_types=[
        pltpu.VMEM((NCHUNK, CH), jnp.int32),
        pltpu.VMEM((NCHUNK, CH), jnp.int32),
        pltpu.VMEM((CH,), jnp.float32),
        pltpu.VMEM((CH, LATDIM), jnp.float32),
        pltpu.VMEM_SHARED((N, LATDIM), jnp.float32),
        pltpu.SemaphoreType.DMA,
    ],
)(_sc_spmm_body)


def _spmm(col2d, row2d, val2d, prev):
    return _sc_spmm(col2d, row2d, val2d, prev)


def _l2n(x):
    nrm = jnp.sqrt(jnp.sum(x * x, axis=1, keepdims=True))
    return x / jnp.maximum(nrm, 1e-12)


def _prep_body(de_ref, dh_ref, ge_ref, gh_ref, ddh_ref, ggh_ref):
    ddh_ref[...] = jnp.dot(de_ref[...], dh_ref[...],
                           preferred_element_type=jnp.float32)
    ggh_ref[...] = jnp.dot(ge_ref[...], gh_ref[...],
                           preferred_element_type=jnp.float32)


def _tc_prep(dE, dHyper, gE, gHyper):
    return pl.pallas_call(
        _prep_body,
        out_shape=(jax.ShapeDtypeStruct((DRUG, LATDIM), jnp.float32),
                   jax.ShapeDtypeStruct((GENE, LATDIM), jnp.float32)),
    )(dE, dHyper, gE, gHyper)


def _hyper_body(ddh_ref, ggh_ref, prev_ref, hyp_ref):
    ddh = ddh_ref[...]
    ggh = ggh_ref[...]
    pd = prev_ref[0:DRUG, :]
    pg = prev_ref[DRUG:N, :]
    td = lax.dot_general(ddh, pd, (((0,), (0,)), ((), ())),
                         preferred_element_type=jnp.float32)
    tg = lax.dot_general(ggh, pg, (((0,), (0,)), ((), ())),
                         preferred_element_type=jnp.float32)
    hyp_ref[0:DRUG, :] = _l2n(jnp.dot(ddh, td,
                                      preferred_element_type=jnp.float32))
    hyp_ref[DRUG:N, :] = _l2n(jnp.dot(ggh, tg,
                                      preferred_element_type=jnp.float32))


def _tc_hyper(ddh, ggh, prev):
    return pl.pallas_call(
        _hyper_body,
        out_shape=jax.ShapeDtypeStruct((N, LATDIM), jnp.float32),
    )(ddh, ggh, prev)


def _combine1_body(p_ref, hyp_ref, gcn_ref, new_ref):
    gcn = _l2n(p_ref[0] + p_ref[1])
    gcn_ref[...] = gcn
    new_ref[...] = gcn + hyp_ref[...]


def _tc_combine1(partials, hyp):
    return pl.pallas_call(
        _combine1_body,
        out_shape=(jax.ShapeDtypeStruct((N, LATDIM), jnp.float32),
                   jax.ShapeDtypeStruct((N, LATDIM), jnp.float32)),
    )(partials, hyp)


def _combine2_body(p_ref, hyp_ref, emb_ref, new1_ref,
                   gcn_ref, new_ref, out_ref):
    gcn = _l2n(p_ref[0] + p_ref[1])
    gcn_ref[...] = gcn
    new2 = gcn + hyp_ref[...]
    new_ref[...] = new2
    out_ref[...] = emb_ref[...] + new1_ref[...] + new2


def _tc_combine2(partials, hyp, embeds, new1):
    return pl.pallas_call(
        _combine2_body,
        out_shape=(jax.ShapeDtypeStruct((N, LATDIM), jnp.float32),
                   jax.ShapeDtypeStruct((N, LATDIM), jnp.float32),
                   jax.ShapeDtypeStruct((N, LATDIM), jnp.float32)),
    )(partials, hyp, embeds, new1)


def kernel(edge_index, adj_values, dEmbeds, gEmbeds, dHyper, gHyper, keepRate):
    del keepRate
    embeds = jnp.concatenate([dEmbeds, gEmbeds], axis=0)
    row2d = edge_index[0].reshape(NW, NCHUNK, CH)
    col2d = edge_index[1].reshape(NW, NCHUNK, CH)
    val2d = adj_values

    ddh, ggh = _tc_prep(dEmbeds, dHyper, gEmbeds, gHyper)

    partials1 = _spmm(col2d, row2d, val2d, embeds)
    hyp1 = _tc_hyper(ddh, ggh, embeds)
    gcn1, new1 = _tc_combine1(partials1, hyp1)

    partials2 = _spmm(col2d, row2d, val2d, new1)
    hyp2 = _tc_hyper(ddh, ggh, new1)
    gcn2, new2, out = _tc_combine2(partials2, hyp2, embeds, new1)

    gcn_stack = jnp.stack([embeds, gcn1, gcn2])
    hyp_stack = jnp.stack([embeds, hyp1, hyp2])
    return (out, gcn_stack, hyp_stack)
